# baseline (speedup 1.0000x reference)
import functools

import jax
import jax.numpy as jnp
from jax import lax
from jax.experimental import pallas as pl
from jax.experimental.pallas import tpu as pltpu
from jax.experimental.pallas import tpu_sc as plsc

N_NODES = 10000
N_PAD = 10240
H = 128
CH = 128
NW = 32
LANES = 16
BN_SCALE = (1.0 + 1e-5) ** -0.5
BLK = 1024
NPR = 3456
ACC_ROWS = NPR + 128
NPASS = -(-N_PAD // NPR)


def _edge_body(nchunk, p_hbm, q_hbm, row_hbm, col_hbm, zero_hbm, r_out,
               ridx, cidx, lidx, qbuf, pbuf, hbuf, acc_sh,
               sem_q, sem_p):
    c = lax.axis_index("c")
    s = lax.axis_index("s")
    wid = c * 16 + s

    pltpu.sync_copy(row_hbm.at[wid], ridx)
    pltpu.sync_copy(col_hbm.at[wid], cidx)

    for npass in range(NPASS):
        base = npass * NPR
        rows_this = min(NPR, N_PAD - base)
        rpt = ACC_ROWS // 16
        pltpu.sync_copy(zero_hbm.at[pl.ds(s * rpt, rpt)],
                        acc_sh.at[pl.ds(s * rpt, rpt)])
        plsc.subcore_barrier()

        def chunk(j, carry, _base=base, _rows=rows_this):
            cp_q = pltpu.async_copy(q_hbm.at[cidx.at[j]], qbuf, sem_q)
            cp_p = pltpu.async_copy(p_hbm.at[ridx.at[j]], pbuf, sem_p)
            for k in range(CH // LANES):
                d = pl.ds(k * LANES, LANES)
                v = ridx[j, d] - _base
                oob = (v < 0) | (v >= _rows)
                lidx[0, d] = jnp.where(oob, NPR, v)
            cp_q.wait()
            cp_p.wait()

            def row(e, carry2):
                for t in range(H // LANES):
                    d = pl.ds(t * LANES, LANES)
                    hbuf[e, d] = jnp.maximum(qbuf[e, d] + pbuf[e, d], 0.0)
                return carry2

            lax.fori_loop(0, CH, row, 0, unroll=2)
            pltpu.sync_copy(hbuf, acc_sh.at[lidx.at[0]], add=True)
            return carry

        lax.fori_loop(0, nchunk, chunk, 0)
        plsc.subcore_barrier()

        rt = rows_this // 16
        pltpu.sync_copy(acc_sh.at[pl.ds(s * rt, rt)],
                        r_out.at[c, pl.ds(base + s * rt, rt)])
        plsc.subcore_barrier()


@functools.lru_cache(maxsize=None)
def _edge_kernel(nchunk):
    mesh = plsc.VectorSubcoreMesh(core_axis_name="c", subcore_axis_name="s",
                                  num_cores=2, num_subcores=16)
    out_type = [jax.ShapeDtypeStruct((2, N_PAD, H), jnp.float32)]
    scratch = [
        pltpu.VMEM((nchunk, CH), jnp.int32),
        pltpu.VMEM((nchunk, CH), jnp.int32),
        pltpu.VMEM((1, CH), jnp.int32),
        pltpu.VMEM((CH, H), jnp.float32),
        pltpu.VMEM((CH, H), jnp.float32),
        pltpu.VMEM((CH, H), jnp.float32),
        pltpu.VMEM_SHARED((ACC_ROWS, H), jnp.float32),
        pltpu.SemaphoreType.DMA, pltpu.SemaphoreType.DMA,
    ]
    return pl.kernel(
        functools.partial(_edge_body, nchunk),
        out_type=out_type,
        mesh=mesh,
        scratch_types=scratch,
    )


def _pq(f, cw_ref, cb_ref, p_ref, q_ref):
    wt = cw_ref[0:H, :]
    wb = cw_ref[H:2 * H, :]
    p_ref[...] = jnp.dot(f, wt - wb, preferred_element_type=jnp.float32) + cb_ref[...]
    q_ref[...] = jnp.dot(f, wb, preferred_element_type=jnp.float32)


def _enc_body(x_ref, w1_ref, b1_ref, w2_ref, b2_ref, cw_ref, cb_ref,
              f_ref, p_ref, q_ref):
    x = x_ref[...]
    h = jnp.maximum(jnp.dot(x, w1_ref[...],
                            preferred_element_type=jnp.float32) + b1_ref[...], 0.0)
    f = jnp.maximum(jnp.dot(h, w2_ref[...],
                            preferred_element_type=jnp.float32) + b2_ref[...], 0.0)
    f_ref[...] = f
    _pq(f, cw_ref, cb_ref, p_ref, q_ref)


def _mid_body(rp_ref, cp_ref, f_ref, g_ref, bb_ref, cw_ref, cb_ref,
              f2_ref, p_ref, q_ref):
    rp = rp_ref[...]
    r = rp[0] + rp[1]
    raw = cp_ref[...][:, 0:1]
    denom = jnp.maximum(raw, 1.0)
    gamma = g_ref[...] * BN_SCALE
    f2 = (r * gamma + raw * bb_ref[...]) / denom + f_ref[...]
    f2_ref[...] = f2
    _pq(f2, cw_ref, cb_ref, p_ref, q_ref)


def _head_body(f_ref, w1_ref, b1_ref, w2_ref, b2_ref, w3_ref, b3_ref, o_ref):
    h1 = jnp.maximum(jnp.dot(f_ref[...], w1_ref[...],
                             preferred_element_type=jnp.float32) + b1_ref[...], 0.0)
    h2 = jnp.maximum(jnp.dot(h1, w2_ref[...],
                             preferred_element_type=jnp.float32) + b2_ref[...], 0.0)
    o_ref[...] = jnp.dot(h2, w3_ref[...],
                         preferred_element_type=jnp.float32) + b3_ref[...]


def _row_spec(cols):
    return pl.BlockSpec((BLK, cols), lambda i: (i, 0))


def _full_spec(shape):
    nd = len(shape)
    return pl.BlockSpec(shape, lambda i: (0,) * nd)


def _enc_call(xp, w1, b1, w2, b2, cw0, cb0):
    return pl.pallas_call(
        _enc_body,
        grid=(N_PAD // BLK,),
        in_specs=[_row_spec(16), _full_spec((16, H)), _full_spec((1, H)),
                  _full_spec((H, H)), _full_spec((1, H)),
                  _full_spec((2 * H, H)), _full_spec((1, H))],
        out_specs=[_row_spec(H), _row_spec(H), _row_spec(H)],
        out_shape=[jax.ShapeDtypeStruct((N_PAD, H), jnp.float32)] * 3,
    )(xp, w1, b1, w2, b2, cw0, cb0)


def _mid_call(rp, cp, f, g, bb, cw, cb):
    return pl.pallas_call(
        _mid_body,
        grid=(N_PAD // BLK,),
        in_specs=[pl.BlockSpec((2, BLK, H), lambda i: (0, i, 0)),
                  _row_spec(LANES), _row_spec(H),
                  _full_spec((1, H)), _full_spec((1, H)),
                  _full_spec((2 * H, H)), _full_spec((1, H))],
        out_specs=[_row_spec(H), _row_spec(H), _row_spec(H)],
        out_shape=[jax.ShapeDtypeStruct((N_PAD, H), jnp.float32)] * 3,
    )(rp, cp, f, g, bb, cw, cb)


def _head_call(f, w1, b1, w2, b2, w3, b3):
    return pl.pallas_call(
        _head_body,
        grid=(N_PAD // BLK,),
        in_specs=[_row_spec(H),
                  _full_spec((H, 64)), _full_spec((1, 64)),
                  _full_spec((64, 32)), _full_spec((1, 32)),
                  _full_spec((32, 8)), _full_spec((1, 8))],
        out_specs=_row_spec(8),
        out_shape=jax.ShapeDtypeStruct((N_PAD, 8), jnp.float32),
    )(f, w1, b1, w2, b2, w3, b3)


def kernel(x, edge_index, batch, lc_w1, lc_b1, lc_w2, lc_b2, conv_w, conv_b,
           bn_g, bn_b, out_w1, out_b1, out_w2, out_b2, out_w3, out_b3):
    e = edge_index.shape[1]
    per_chunk = NW * CH
    nchunk = -(-e // per_chunk)
    e_pad = nchunk * per_chunk

    row = edge_index[0].astype(jnp.int32)
    col = edge_index[1].astype(jnp.int32)
    pad_idx = jnp.full((e_pad - e,), N_NODES, dtype=jnp.int32)
    row3 = jnp.concatenate([row, pad_idx]).reshape(NW, nchunk, CH)
    col3 = jnp.concatenate([col, pad_idx]).reshape(NW, nchunk, CH)

    xp = jnp.pad(x, ((0, N_PAD - x.shape[0]), (0, 0)))
    zero = jnp.zeros((ACC_ROWS, H), jnp.float32)
    ones_pq = jnp.ones((N_PAD, H), jnp.float32)
    zeros_pq = jnp.zeros((N_PAD, H), jnp.float32)

    feats, p, q = _enc_call(xp, lc_w1, lc_b1[None, :], lc_w2, lc_b2[None, :],
                            conv_w[0], conv_b[0][None, :])

    ek = _edge_kernel(nchunk)
    cw_roll = jnp.roll(conv_w, -1, axis=0)
    cb_roll = jnp.roll(conv_b, -1, axis=0)
    n_layers = conv_w.shape[0]
    bound = n_layers + 1 + jnp.min(batch).astype(jnp.int32)

    def step(carry):
        i, f, pl_, ql_, cnt = carry
        first = i == 0
        p_in = jnp.where(first, ones_pq, pl_)
        q_in = jnp.where(first, zeros_pq, ql_)
        (rp,) = ek(p_in, q_in, row3, col3, zero)
        cnt = jnp.where(first, rp[0, :, :LANES] + rp[1, :, :LANES], cnt)
        li = jnp.maximum(i - 1, 0)
        f2, p2, q2 = _mid_call(rp, cnt, f,
                               lax.dynamic_index_in_dim(bn_g, li, 0),
                               lax.dynamic_index_in_dim(bn_b, li, 0),
                               lax.dynamic_index_in_dim(cw_roll, li, 0,
                                                        keepdims=False),
                               lax.dynamic_index_in_dim(cb_roll, li, 0))
        f = jnp.where(first, f, f2)
        pl_ = jnp.where(first, pl_, p2)
        ql_ = jnp.where(first, ql_, q2)
        return (i + 1, f, pl_, ql_, cnt)

    cnt0 = jnp.zeros((N_PAD, LANES), jnp.float32)
    _, feats, _, _, _ = lax.while_loop(lambda cy: cy[0] < bound, step,
                                       (jnp.int32(0), feats, p, q, cnt0))

    out = _head_call(feats, out_w1, out_b1[None, :], out_w2, out_b2[None, :],
                     out_w3, out_b3[None, :])
    return (out[:N_NODES], batch)


# speedup vs baseline: 1.3697x; 1.3697x over previous
import functools

import jax
import jax.numpy as jnp
from jax import lax
from jax.experimental import pallas as pl
from jax.experimental.pallas import tpu as pltpu
from jax.experimental.pallas import tpu_sc as plsc

N_NODES = 10000
N_PAD = 10240
H = 128
CH = 128
NW = 32
LANES = 16
BN_SCALE = (1.0 + 1e-5) ** -0.5
BLK = 1024
NPR = 3456
ACC_ROWS = NPR + 16
NPASS = -(-N_PAD // NPR)


def _edge_body(nchunk, p_hbm, q_hbm, row_hbm, col_hbm, zero_hbm, r_out,
               ridx, cidx, lidx, qbuf, pbuf, hbuf, acc_sh,
               sq0, sq1):
    c = lax.axis_index("c")
    s = lax.axis_index("s")
    wid = c * 16 + s
    sq = (sq0, sq1)

    pltpu.sync_copy(row_hbm.at[wid], ridx)
    pltpu.sync_copy(col_hbm.at[wid], cidx)

    def issue_gather(jj, b):
        pltpu.async_copy(q_hbm.at[cidx.at[jj]], qbuf.at[b], sq[b])
        pltpu.async_copy(p_hbm.at[ridx.at[jj]], pbuf.at[b], sq[b])

    def wait_gather(b):
        pltpu.make_async_copy(q_hbm.at[pl.ds(0, CH)], qbuf.at[b], sq[b]).wait()
        pltpu.make_async_copy(p_hbm.at[pl.ds(0, CH)], pbuf.at[b], sq[b]).wait()

    for npass in range(NPASS):
        base = npass * NPR
        rows_this = min(NPR, N_PAD - base)
        rpt = NPR // 16
        pltpu.sync_copy(zero_hbm.at[pl.ds(s * rpt, rpt)],
                        acc_sh.at[pl.ds(s * rpt, rpt)])
        plsc.subcore_barrier()
        issue_gather(0, 0)
        issue_gather(1, 1)

        def pair(j2, carry, _base=base, _rows=rows_this):
            for b in (0, 1):
                jj = 2 * j2 + b
                for k in range(CH // LANES):
                    d = pl.ds(k * LANES, LANES)
                    v = ridx[jj, d] - _base
                    oob = (v < 0) | (v >= _rows)
                    lidx[0, d] = jnp.where(oob, NPR, v)
                wait_gather(b)
                qb = qbuf.at[b]
                pb = pbuf.at[b]

                def row(e, carry2):
                    for t in range(H // LANES):
                        d = pl.ds(t * LANES, LANES)
                        hbuf[e, d] = jnp.maximum(qb[e, d] + pb[e, d], 0.0)
                    return carry2

                lax.fori_loop(0, CH, row, 0, unroll=2)

                @pl.when(jj + 2 < nchunk)
                def _(jj=jj, b=b):
                    issue_gather(jj + 2, b)

                pltpu.sync_copy(hbuf, acc_sh.at[lidx.at[0]], add=True)
            return carry

        lax.fori_loop(0, nchunk // 2, pair, 0)
        plsc.subcore_barrier()

        rt = rows_this // 16
        pltpu.sync_copy(acc_sh.at[pl.ds(s * rt, rt)],
                        r_out.at[c, pl.ds(base + s * rt, rt)])
        plsc.subcore_barrier()


@functools.lru_cache(maxsize=None)
def _edge_kernel(nchunk):
    mesh = plsc.VectorSubcoreMesh(core_axis_name="c", subcore_axis_name="s",
                                  num_cores=2, num_subcores=16)
    out_type = [jax.ShapeDtypeStruct((2, N_PAD, H), jnp.float32)]
    scratch = [
        pltpu.VMEM((nchunk, CH), jnp.int32),
        pltpu.VMEM((nchunk, CH), jnp.int32),
        pltpu.VMEM((1, CH), jnp.int32),
        pltpu.VMEM((2, CH, H), jnp.float32),
        pltpu.VMEM((2, CH, H), jnp.float32),
        pltpu.VMEM((CH, H), jnp.float32),
        pltpu.VMEM_SHARED((ACC_ROWS, H), jnp.float32),
    ] + [pltpu.SemaphoreType.DMA] * 2
    return pl.kernel(
        functools.partial(_edge_body, nchunk),
        out_type=out_type,
        mesh=mesh,
        scratch_types=scratch,
    )


def _pq(f, cw_ref, cb_ref, p_ref, q_ref):
    wt = cw_ref[0:H, :]
    wb = cw_ref[H:2 * H, :]
    p_ref[...] = jnp.dot(f, wt - wb, preferred_element_type=jnp.float32) + cb_ref[...]
    q_ref[...] = jnp.dot(f, wb, preferred_element_type=jnp.float32)


def _enc_body(x_ref, w1_ref, b1_ref, w2_ref, b2_ref, cw_ref, cb_ref,
              f_ref, p_ref, q_ref):
    x = x_ref[...]
    h = jnp.maximum(jnp.dot(x, w1_ref[...],
                            preferred_element_type=jnp.float32) + b1_ref[...], 0.0)
    f = jnp.maximum(jnp.dot(h, w2_ref[...],
                            preferred_element_type=jnp.float32) + b2_ref[...], 0.0)
    f_ref[...] = f
    _pq(f, cw_ref, cb_ref, p_ref, q_ref)


def _mid_body(rp_ref, cp_ref, f_ref, g_ref, bb_ref, cw_ref, cb_ref,
              f2_ref, p_ref, q_ref):
    rp = rp_ref[...]
    r = rp[0] + rp[1]
    raw = cp_ref[...][:, 0:1]
    denom = jnp.maximum(raw, 1.0)
    gamma = g_ref[...] * BN_SCALE
    f2 = (r * gamma + raw * bb_ref[...]) / denom + f_ref[...]
    f2_ref[...] = f2
    _pq(f2, cw_ref, cb_ref, p_ref, q_ref)


def _head_body(f_ref, w1_ref, b1_ref, w2_ref, b2_ref, w3_ref, b3_ref, o_ref):
    h1 = jnp.maximum(jnp.dot(f_ref[...], w1_ref[...],
                             preferred_element_type=jnp.float32) + b1_ref[...], 0.0)
    h2 = jnp.maximum(jnp.dot(h1, w2_ref[...],
                             preferred_element_type=jnp.float32) + b2_ref[...], 0.0)
    o_ref[...] = jnp.dot(h2, w3_ref[...],
                         preferred_element_type=jnp.float32) + b3_ref[...]


def _row_spec(cols):
    return pl.BlockSpec((BLK, cols), lambda i: (i, 0))


def _full_spec(shape):
    nd = len(shape)
    return pl.BlockSpec(shape, lambda i: (0,) * nd)


def _enc_call(xp, w1, b1, w2, b2, cw0, cb0):
    return pl.pallas_call(
        _enc_body,
        grid=(N_PAD // BLK,),
        in_specs=[_row_spec(16), _full_spec((16, H)), _full_spec((1, H)),
                  _full_spec((H, H)), _full_spec((1, H)),
                  _full_spec((2 * H, H)), _full_spec((1, H))],
        out_specs=[_row_spec(H), _row_spec(H), _row_spec(H)],
        out_shape=[jax.ShapeDtypeStruct((N_PAD, H), jnp.float32)] * 3,
    )(xp, w1, b1, w2, b2, cw0, cb0)


def _mid_call(rp, cp, f, g, bb, cw, cb):
    return pl.pallas_call(
        _mid_body,
        grid=(N_PAD // BLK,),
        in_specs=[pl.BlockSpec((2, BLK, H), lambda i: (0, i, 0)),
                  _row_spec(LANES), _row_spec(H),
                  _full_spec((1, H)), _full_spec((1, H)),
                  _full_spec((2 * H, H)), _full_spec((1, H))],
        out_specs=[_row_spec(H), _row_spec(H), _row_spec(H)],
        out_shape=[jax.ShapeDtypeStruct((N_PAD, H), jnp.float32)] * 3,
    )(rp, cp, f, g, bb, cw, cb)


def _head_call(f, w1, b1, w2, b2, w3, b3):
    return pl.pallas_call(
        _head_body,
        grid=(N_PAD // BLK,),
        in_specs=[_row_spec(H),
                  _full_spec((H, 64)), _full_spec((1, 64)),
                  _full_spec((64, 32)), _full_spec((1, 32)),
                  _full_spec((32, 8)), _full_spec((1, 8))],
        out_specs=_row_spec(8),
        out_shape=jax.ShapeDtypeStruct((N_PAD, 8), jnp.float32),
    )(f, w1, b1, w2, b2, w3, b3)


def kernel(x, edge_index, batch, lc_w1, lc_b1, lc_w2, lc_b2, conv_w, conv_b,
           bn_g, bn_b, out_w1, out_b1, out_w2, out_b2, out_w3, out_b3):
    e = edge_index.shape[1]
    per_chunk = NW * CH
    nchunk = -(-e // per_chunk)
    nchunk += nchunk % 2
    e_pad = nchunk * per_chunk

    row = edge_index[0].astype(jnp.int32)
    col = edge_index[1].astype(jnp.int32)
    pad_idx = jnp.full((e_pad - e,), N_NODES, dtype=jnp.int32)
    row3 = jnp.concatenate([row, pad_idx]).reshape(NW, nchunk, CH)
    col3 = jnp.concatenate([col, pad_idx]).reshape(NW, nchunk, CH)

    xp = jnp.pad(x, ((0, N_PAD - x.shape[0]), (0, 0)))
    zero = jnp.zeros((NPR, H), jnp.float32)
    ones_pq = jnp.ones((N_PAD, H), jnp.float32)
    zeros_pq = jnp.zeros((N_PAD, H), jnp.float32)

    feats, p, q = _enc_call(xp, lc_w1, lc_b1[None, :], lc_w2, lc_b2[None, :],
                            conv_w[0], conv_b[0][None, :])

    ek = _edge_kernel(nchunk)
    cw_roll = jnp.roll(conv_w, -1, axis=0)
    cb_roll = jnp.roll(conv_b, -1, axis=0)
    n_layers = conv_w.shape[0]
    bound = n_layers + 1 + jnp.min(batch).astype(jnp.int32)

    def step(carry):
        i, f, pl_, ql_, cnt = carry
        first = i == 0
        p_in = jnp.where(first, ones_pq, pl_)
        q_in = jnp.where(first, zeros_pq, ql_)
        (rp,) = ek(p_in, q_in, row3, col3, zero)
        cnt = jnp.where(first, rp[0, :, :LANES] + rp[1, :, :LANES], cnt)
        li = jnp.maximum(i - 1, 0)
        f2, p2, q2 = _mid_call(rp, cnt, f,
                               lax.dynamic_index_in_dim(bn_g, li, 0),
                               lax.dynamic_index_in_dim(bn_b, li, 0),
                               lax.dynamic_index_in_dim(cw_roll, li, 0,
                                                        keepdims=False),
                               lax.dynamic_index_in_dim(cb_roll, li, 0))
        f = jnp.where(first, f, f2)
        pl_ = jnp.where(first, pl_, p2)
        ql_ = jnp.where(first, ql_, q2)
        return (i + 1, f, pl_, ql_, cnt)

    cnt0 = jnp.zeros((N_PAD, LANES), jnp.float32)
    _, feats, _, _, _ = lax.while_loop(lambda cy: cy[0] < bound, step,
                                       (jnp.int32(0), feats, p, q, cnt0))

    out = _head_call(feats, out_w1, out_b1[None, :], out_w2, out_b2[None, :],
                     out_w3, out_b3[None, :])
    return (out[:N_NODES], batch)


# speedup vs baseline: 1.6225x; 1.1845x over previous
import functools

import jax
import jax.numpy as jnp
from jax import lax
from jax.experimental import pallas as pl
from jax.experimental.pallas import tpu as pltpu
from jax.experimental.pallas import tpu_sc as plsc

N_NODES = 10000
N_PAD = 10240
H = 128
CH = 128
NW = 32
LANES = 16
BN_SCALE = (1.0 + 1e-5) ** -0.5
BLK = 1024
NPR = 3456
ACC_ROWS = NPR + 16
NPASS = -(-N_PAD // NPR)


def _edge_body(nchunk, p_hbm, q_hbm, row_hbm, col_hbm, zero_hbm, flag_hbm,
               r_out,
               ridx, cidx, lidx, fbuf, qbuf, pbuf, hbuf, acc_sh,
               sq0, sq1):
    c = lax.axis_index("c")
    s = lax.axis_index("s")
    wid = c * 16 + s
    sq = (sq0, sq1)

    pltpu.sync_copy(row_hbm.at[wid], ridx)
    pltpu.sync_copy(col_hbm.at[wid], cidx)
    pltpu.sync_copy(flag_hbm, fbuf)
    is_cnt = fbuf[0, pl.ds(0, LANES)][0] == 1

    @pl.when(is_cnt)
    def _():
        def fill(e, carry2):
            for t in range(H // LANES):
                hbuf[e, pl.ds(t * LANES, LANES)] = jnp.full(
                    (LANES,), 1.0, jnp.float32)
            return carry2
        lax.fori_loop(0, CH, fill, 0, unroll=2)

    def issue_gather(jj, b):
        pltpu.async_copy(q_hbm.at[cidx.at[jj]], qbuf.at[b], sq[b])
        pltpu.async_copy(p_hbm.at[ridx.at[jj]], pbuf.at[b], sq[b])

    def wait_gather(b):
        pltpu.make_async_copy(q_hbm.at[pl.ds(0, CH)], qbuf.at[b], sq[b]).wait()
        pltpu.make_async_copy(p_hbm.at[pl.ds(0, CH)], pbuf.at[b], sq[b]).wait()

    for npass in range(NPASS):
        base = npass * NPR
        rows_this = min(NPR, N_PAD - base)
        rpt = NPR // 16
        pltpu.sync_copy(zero_hbm.at[pl.ds(s * rpt, rpt)],
                        acc_sh.at[pl.ds(s * rpt, rpt)])
        plsc.subcore_barrier()

        @pl.when(jnp.logical_not(is_cnt))
        def _():
            issue_gather(0, 0)
            issue_gather(1, 1)

        def pair(j2, carry, _base=base, _rows=rows_this):
            for b in (0, 1):
                jj = 2 * j2 + b
                for k in range(CH // LANES):
                    d = pl.ds(k * LANES, LANES)
                    v = ridx[jj, d] - _base
                    oob = (v < 0) | (v >= _rows)
                    lidx[0, d] = jnp.where(oob, NPR, v)

                @pl.when(jnp.logical_not(is_cnt))
                def _(jj=jj, b=b):
                    wait_gather(b)
                    qb = qbuf.at[b]
                    pb = pbuf.at[b]

                    def row(e, carry2):
                        for t in range(H // LANES):
                            d = pl.ds(t * LANES, LANES)
                            hbuf[e, d] = jnp.maximum(qb[e, d] + pb[e, d], 0.0)
                        return carry2

                    lax.fori_loop(0, CH, row, 0, unroll=2)

                    @pl.when(jj + 2 < nchunk)
                    def _():
                        issue_gather(jj + 2, b)

                pltpu.sync_copy(hbuf, acc_sh.at[lidx.at[0]], add=True)
            return carry

        lax.fori_loop(0, nchunk // 2, pair, 0)
        plsc.subcore_barrier()

        rt = rows_this // 16
        pltpu.sync_copy(acc_sh.at[pl.ds(s * rt, rt)],
                        r_out.at[c, pl.ds(base + s * rt, rt)])
        plsc.subcore_barrier()


@functools.lru_cache(maxsize=None)
def _edge_kernel(nchunk):
    mesh = plsc.VectorSubcoreMesh(core_axis_name="c", subcore_axis_name="s",
                                  num_cores=2, num_subcores=16)
    out_type = [jax.ShapeDtypeStruct((2, N_PAD, H), jnp.float32)]
    scratch = [
        pltpu.VMEM((nchunk, CH), jnp.int32),
        pltpu.VMEM((nchunk, CH), jnp.int32),
        pltpu.VMEM((1, CH), jnp.int32),
        pltpu.VMEM((1, CH), jnp.int32),
        pltpu.VMEM((2, CH, H), jnp.float32),
        pltpu.VMEM((2, CH, H), jnp.float32),
        pltpu.VMEM((CH, H), jnp.float32),
        pltpu.VMEM_SHARED((ACC_ROWS, H), jnp.float32),
    ] + [pltpu.SemaphoreType.DMA] * 2
    return pl.kernel(
        functools.partial(_edge_body, nchunk),
        out_type=out_type,
        mesh=mesh,
        scratch_types=scratch,
    )


def _pq(f, cw_ref, cb_ref, p_ref, q_ref):
    wt = cw_ref[0:H, :]
    wb = cw_ref[H:2 * H, :]
    p_ref[...] = jnp.dot(f, wt - wb, preferred_element_type=jnp.float32) + cb_ref[...]
    q_ref[...] = jnp.dot(f, wb, preferred_element_type=jnp.float32)


def _enc_body(x_ref, w1_ref, b1_ref, w2_ref, b2_ref, cw_ref, cb_ref,
              f_ref, p_ref, q_ref):
    x = x_ref[...]
    h = jnp.maximum(jnp.dot(x, w1_ref[...],
                            preferred_element_type=jnp.float32) + b1_ref[...], 0.0)
    f = jnp.maximum(jnp.dot(h, w2_ref[...],
                            preferred_element_type=jnp.float32) + b2_ref[...], 0.0)
    f_ref[...] = f
    _pq(f, cw_ref, cb_ref, p_ref, q_ref)


def _mid_body(rp_ref, cp_ref, f_ref, g_ref, bb_ref, cw_ref, cb_ref,
              f2_ref, p_ref, q_ref):
    rp = rp_ref[...]
    r = rp[0] + rp[1]
    raw = cp_ref[...][:, 0:1]
    denom = jnp.maximum(raw, 1.0)
    gamma = g_ref[...] * BN_SCALE
    f2 = (r * gamma + raw * bb_ref[...]) / denom + f_ref[...]
    f2_ref[...] = f2
    _pq(f2, cw_ref, cb_ref, p_ref, q_ref)


def _head_body(f_ref, w1_ref, b1_ref, w2_ref, b2_ref, w3_ref, b3_ref, o_ref):
    h1 = jnp.maximum(jnp.dot(f_ref[...], w1_ref[...],
                             preferred_element_type=jnp.float32) + b1_ref[...], 0.0)
    h2 = jnp.maximum(jnp.dot(h1, w2_ref[...],
                             preferred_element_type=jnp.float32) + b2_ref[...], 0.0)
    o_ref[...] = jnp.dot(h2, w3_ref[...],
                         preferred_element_type=jnp.float32) + b3_ref[...]


def _row_spec(cols):
    return pl.BlockSpec((BLK, cols), lambda i: (i, 0))


def _full_spec(shape):
    nd = len(shape)
    return pl.BlockSpec(shape, lambda i: (0,) * nd)


def _enc_call(xp, w1, b1, w2, b2, cw0, cb0):
    return pl.pallas_call(
        _enc_body,
        grid=(N_PAD // BLK,),
        in_specs=[_row_spec(16), _full_spec((16, H)), _full_spec((1, H)),
                  _full_spec((H, H)), _full_spec((1, H)),
                  _full_spec((2 * H, H)), _full_spec((1, H))],
        out_specs=[_row_spec(H), _row_spec(H), _row_spec(H)],
        out_shape=[jax.ShapeDtypeStruct((N_PAD, H), jnp.float32)] * 3,
    )(xp, w1, b1, w2, b2, cw0, cb0)


def _mid_call(rp, cp, f, g, bb, cw, cb):
    return pl.pallas_call(
        _mid_body,
        grid=(N_PAD // BLK,),
        in_specs=[pl.BlockSpec((2, BLK, H), lambda i: (0, i, 0)),
                  _row_spec(LANES), _row_spec(H),
                  _full_spec((1, H)), _full_spec((1, H)),
                  _full_spec((2 * H, H)), _full_spec((1, H))],
        out_specs=[_row_spec(H), _row_spec(H), _row_spec(H)],
        out_shape=[jax.ShapeDtypeStruct((N_PAD, H), jnp.float32)] * 3,
    )(rp, cp, f, g, bb, cw, cb)


def _head_call(f, w1, b1, w2, b2, w3, b3):
    return pl.pallas_call(
        _head_body,
        grid=(N_PAD // BLK,),
        in_specs=[_row_spec(H),
                  _full_spec((H, 64)), _full_spec((1, 64)),
                  _full_spec((64, 32)), _full_spec((1, 32)),
                  _full_spec((32, 8)), _full_spec((1, 8))],
        out_specs=_row_spec(8),
        out_shape=jax.ShapeDtypeStruct((N_PAD, 8), jnp.float32),
    )(f, w1, b1, w2, b2, w3, b3)


def kernel(x, edge_index, batch, lc_w1, lc_b1, lc_w2, lc_b2, conv_w, conv_b,
           bn_g, bn_b, out_w1, out_b1, out_w2, out_b2, out_w3, out_b3):
    e = edge_index.shape[1]
    per_chunk = NW * CH
    nchunk = -(-e // per_chunk)
    nchunk += nchunk % 2
    e_pad = nchunk * per_chunk

    row = edge_index[0].astype(jnp.int32)
    col = edge_index[1].astype(jnp.int32)
    pad_idx = jnp.full((e_pad - e,), N_NODES, dtype=jnp.int32)
    row3 = jnp.concatenate([row, pad_idx]).reshape(NW, nchunk, CH)
    col3 = jnp.concatenate([col, pad_idx]).reshape(NW, nchunk, CH)

    xp = jnp.pad(x, ((0, N_PAD - x.shape[0]), (0, 0)))
    zero = jnp.zeros((NPR, H), jnp.float32)

    feats, p, q = _enc_call(xp, lc_w1, lc_b1[None, :], lc_w2, lc_b2[None, :],
                            conv_w[0], conv_b[0][None, :])

    ek = _edge_kernel(nchunk)
    cw_roll = jnp.roll(conv_w, -1, axis=0)
    cb_roll = jnp.roll(conv_b, -1, axis=0)
    n_layers = conv_w.shape[0]
    bound = n_layers + 1 + jnp.min(batch).astype(jnp.int32)

    def step(carry):
        i, f, pl_, ql_, cnt = carry
        first = i == 0
        flag = first.astype(jnp.int32) * jnp.ones((1, CH), jnp.int32)
        (rp,) = ek(pl_, ql_, row3, col3, zero, flag)
        cnt = jnp.where(first, rp[0, :, :LANES] + rp[1, :, :LANES], cnt)
        li = jnp.maximum(i - 1, 0)
        f2, p2, q2 = _mid_call(rp, cnt, f,
                               lax.dynamic_index_in_dim(bn_g, li, 0),
                               lax.dynamic_index_in_dim(bn_b, li, 0),
                               lax.dynamic_index_in_dim(cw_roll, li, 0,
                                                        keepdims=False),
                               lax.dynamic_index_in_dim(cb_roll, li, 0))
        f = jnp.where(first, f, f2)
        pl_ = jnp.where(first, pl_, p2)
        ql_ = jnp.where(first, ql_, q2)
        return (i + 1, f, pl_, ql_, cnt)

    cnt0 = jnp.zeros((N_PAD, LANES), jnp.float32)
    _, feats, _, _, _ = lax.while_loop(lambda cy: cy[0] < bound, step,
                                       (jnp.int32(0), feats, p, q, cnt0))

    out = _head_call(feats, out_w1, out_b1[None, :], out_w2, out_b2[None, :],
                     out_w3, out_b3[None, :])
    return (out[:N_NODES], batch)
